# baseline (speedup 1.0000x reference)
import jax
import jax.numpy as jnp
from jax.experimental import pallas as pl
from jax.experimental.pallas import tpu as pltpu

INPUT_DIM = 1024
OUTPUT_DIM = 1024
NUM_EXPERTS = 8
TOP_K = 2
BATCH = 2048


def _moe_body(x_ref, wr_ref, br_ref, we_ref, be_ref,
              out_ref, logits_ref, idx_ref, gates_ref):
    e = pl.program_id(0)
    m = pl.program_id(1)
    BM = out_ref.shape[1]

    @pl.when(jnp.logical_and(e == 0, m == 0))
    def _router():
        x = x_ref[...]
        wr = wr_ref[...]
        logits = jax.lax.dot_general(
            x, wr, (((1,), (1,)), ((), ())),
            preferred_element_type=jnp.float32)
        logits = logits + br_ref[...]
        logits_ref[...] = logits

        e_iota = jax.lax.broadcasted_iota(jnp.int32, logits.shape, 1)
        big = jnp.int32(NUM_EXPERTS)
        m1 = jnp.max(logits, axis=1, keepdims=True)
        i1 = jnp.min(jnp.where(logits == m1, e_iota, big), axis=1,
                     keepdims=True)
        masked = jnp.where(e_iota == i1, -jnp.inf, logits)
        m2 = jnp.max(masked, axis=1, keepdims=True)
        i2 = jnp.min(jnp.where(masked == m2, e_iota, big), axis=1,
                     keepdims=True)
        idx_ref[...] = jnp.concatenate([i1, i2], axis=1)
        gates_ref[...] = jnp.where(
            e_iota == i1, m1, jnp.where(e_iota == i2, m2, 0.0))

    x = x_ref[pl.ds(m * BM, BM), :].astype(jnp.bfloat16)
    w = we_ref[0].astype(jnp.bfloat16)
    acc = jax.lax.dot_general(
        x, w, (((1,), (1,)), ((), ())),
        preferred_element_type=jnp.float32)
    acc = jnp.maximum(acc + be_ref[0], 0.0)
    gates = gates_ref[pl.ds(m * BM, BM), :]
    col = jax.lax.broadcasted_iota(jnp.int32, gates.shape, 1)
    g = jnp.sum(jnp.where(col == e, gates, 0.0), axis=1, keepdims=True)
    out_ref[0] = acc * g


def kernel(x, Wr, br, We, be):
    B, I = x.shape
    E, O, _ = We.shape
    BM = 256
    MT = B // BM
    out, logits, idx = pl.pallas_call(
        _moe_body,
        grid=(E, MT),
        in_specs=[
            pl.BlockSpec((B, I), lambda e, m: (0, 0)),
            pl.BlockSpec((E, I), lambda e, m: (0, 0)),
            pl.BlockSpec((1, E), lambda e, m: (0, 0)),
            pl.BlockSpec((1, O, I), lambda e, m: (e, 0, 0)),
            pl.BlockSpec((1, 1, O), lambda e, m: (e, 0, 0)),
        ],
        out_specs=[
            pl.BlockSpec((1, BM, O), lambda e, m: (e, m, 0)),
            pl.BlockSpec((B, E), lambda e, m: (0, 0)),
            pl.BlockSpec((B, TOP_K), lambda e, m: (0, 0)),
        ],
        out_shape=[
            jax.ShapeDtypeStruct((E, B, O), jnp.float32),
            jax.ShapeDtypeStruct((B, E), jnp.float32),
            jax.ShapeDtypeStruct((B, TOP_K), jnp.int32),
        ],
        scratch_shapes=[pltpu.VMEM((B, NUM_EXPERTS), jnp.float32)],
    )(x, Wr, br.reshape(1, E), We, be.reshape(E, 1, O))
    return (out, logits, idx)


# speedup vs baseline: 1.6039x; 1.6039x over previous
import jax
import jax.numpy as jnp
from jax.experimental import pallas as pl
from jax.experimental.pallas import tpu as pltpu

INPUT_DIM = 1024
OUTPUT_DIM = 1024
NUM_EXPERTS = 8
TOP_K = 2
BATCH = 2048


def _moe_body(x_ref, wr_ref, br_ref, we_ref, be_ref,
              out_ref, logits_ref, idx_ref, gates_ref):
    e = pl.program_id(0)

    @pl.when(e == 0)
    def _router():
        x = x_ref[...]
        wr = wr_ref[...]
        logits = jax.lax.dot_general(
            x, wr, (((1,), (1,)), ((), ())),
            preferred_element_type=jnp.float32)
        logits = logits + br_ref[...]
        logits_ref[...] = logits

        e_iota = jax.lax.broadcasted_iota(jnp.int32, logits.shape, 1)
        big = jnp.int32(NUM_EXPERTS)
        m1 = jnp.max(logits, axis=1, keepdims=True)
        i1 = jnp.min(jnp.where(logits == m1, e_iota, big), axis=1,
                     keepdims=True)
        masked = jnp.where(e_iota == i1, -jnp.inf, logits)
        m2 = jnp.max(masked, axis=1, keepdims=True)
        i2 = jnp.min(jnp.where(masked == m2, e_iota, big), axis=1,
                     keepdims=True)
        idx_ref[...] = jnp.concatenate([i1, i2], axis=1)
        gates_ref[...] = jnp.where(
            e_iota == i1, m1, jnp.where(e_iota == i2, m2, 0.0))

    x = x_ref[...]
    w = we_ref[0]
    acc = jax.lax.dot_general(
        x, w, (((1,), (1,)), ((), ())),
        preferred_element_type=jnp.float32)
    acc = jnp.maximum(acc + be_ref[0], 0.0)
    gates = gates_ref[...]
    col = jax.lax.broadcasted_iota(jnp.int32, gates.shape, 1)
    g = jnp.sum(jnp.where(col == e, gates, 0.0), axis=1, keepdims=True)
    out_ref[0] = acc * g


def kernel(x, Wr, br, We, be):
    B, I = x.shape
    E, O, _ = We.shape
    out, logits, idx = pl.pallas_call(
        _moe_body,
        grid=(E,),
        in_specs=[
            pl.BlockSpec((B, I), lambda e: (0, 0)),
            pl.BlockSpec((E, I), lambda e: (0, 0)),
            pl.BlockSpec((1, E), lambda e: (0, 0)),
            pl.BlockSpec((1, O, I), lambda e: (e, 0, 0)),
            pl.BlockSpec((1, 1, O), lambda e: (e, 0, 0)),
        ],
        out_specs=[
            pl.BlockSpec((1, B, O), lambda e: (e, 0, 0)),
            pl.BlockSpec((B, E), lambda e: (0, 0)),
            pl.BlockSpec((B, TOP_K), lambda e: (0, 0)),
        ],
        out_shape=[
            jax.ShapeDtypeStruct((E, B, O), jnp.float32),
            jax.ShapeDtypeStruct((B, E), jnp.float32),
            jax.ShapeDtypeStruct((B, TOP_K), jnp.int32),
        ],
        scratch_shapes=[pltpu.VMEM((B, NUM_EXPERTS), jnp.float32)],
    )(x, Wr, br.reshape(1, E), We, be.reshape(E, 1, O))
    return (out, logits, idx)
